# baseline (speedup 1.0000x reference)
import functools

import jax
import jax.numpy as jnp
from jax import lax
from jax.experimental import pallas as pl
from jax.experimental.pallas import tpu as pltpu
from jax.experimental.pallas import tpu_sc as plsc

_B = 16384
_D = 128
_NC = 2
_NS = 16
_NW = _NC * _NS
_BPW = _B // _NW

_mesh = plsc.VectorSubcoreMesh(core_axis_name="c", subcore_axis_name="s")


_CH = 4
_CR = _BPW // _CH


@functools.partial(
    pl.kernel,
    out_type=jax.ShapeDtypeStruct((_B, _D), jnp.float32),
    mesh=_mesh,
    scratch_types=[
        pltpu.VMEM((_BPW,), jnp.int32),
        pltpu.VMEM((_CH, _CR, _D), jnp.float32),
        [pltpu.SemaphoreType.DMA] * _CH,
        pltpu.SemaphoreType.DMA,
    ],
)
def _emb_lookup(idx_hbm, table_hbm, out_hbm, idx_v, rows_v, gsems, ssem):
    wid = lax.axis_index("s") * _NC + lax.axis_index("c")
    base = wid * _BPW
    pltpu.sync_copy(idx_hbm.at[pl.ds(base, _BPW)], idx_v)
    gathers = [
        pltpu.async_copy(
            table_hbm.at[idx_v.at[pl.ds(c * _CR, _CR)]], rows_v.at[c], gsems[c]
        )
        for c in range(_CH)
    ]
    stores = []
    for c in range(_CH):
        gathers[c].wait()
        stores.append(
            pltpu.async_copy(
                rows_v.at[c], out_hbm.at[pl.ds(base + c * _CR, _CR)], ssem
            )
        )
    for cp in stores:
        cp.wait()


def kernel(x, table):
    idx = x.reshape(-1).astype(jnp.int32)
    return _emb_lookup(idx, table)


# speedup vs baseline: 1.5495x; 1.5495x over previous
import functools

import jax
import jax.numpy as jnp
from jax import lax
from jax.experimental import pallas as pl
from jax.experimental.pallas import tpu as pltpu
from jax.experimental.pallas import tpu_sc as plsc

_B = 16384
_D = 128
_NC = 2
_NS = 16
_NW = _NC * _NS
_BPW = _B // _NW

_mesh = plsc.VectorSubcoreMesh(core_axis_name="c", subcore_axis_name="s")


_CH = 4
_CR = _BPW // _CH
_V = 100


@functools.partial(
    pl.kernel,
    out_type=jax.ShapeDtypeStruct((_B, _D), jnp.float32),
    mesh=_mesh,
    scratch_types=[
        pltpu.VMEM((_BPW,), jnp.int32),
        pltpu.VMEM((_CH, _CR, _D), jnp.float32),
        pltpu.VMEM_SHARED((_V, _D), jnp.float32),
        [pltpu.SemaphoreType.DMA] * _CH,
        pltpu.SemaphoreType.DMA,
    ],
)
def _emb_lookup(idx_hbm, table_hbm, out_hbm, idx_v, rows_v, table_sh, gsems, ssem):
    sid = lax.axis_index("s")
    wid = sid * _NC + lax.axis_index("c")
    base = wid * _BPW
    @pl.when(sid == 0)
    def _():
        pltpu.sync_copy(table_hbm, table_sh)

    pltpu.sync_copy(idx_hbm.at[pl.ds(base, _BPW)], idx_v)
    plsc.subcore_barrier()
    gathers = [
        pltpu.async_copy(
            table_sh.at[idx_v.at[pl.ds(c * _CR, _CR)]], rows_v.at[c], gsems[c]
        )
        for c in range(_CH)
    ]
    stores = []
    for c in range(_CH):
        gathers[c].wait()
        stores.append(
            pltpu.async_copy(
                rows_v.at[c], out_hbm.at[pl.ds(base + c * _CR, _CR)], ssem
            )
        )
    for cp in stores:
        cp.wait()


def kernel(x, table):
    idx = x.reshape(-1).astype(jnp.int32)
    return _emb_lookup(idx, table)


# speedup vs baseline: 1.6015x; 1.0335x over previous
import functools

import jax
import jax.numpy as jnp
from jax import lax
from jax.experimental import pallas as pl
from jax.experimental.pallas import tpu as pltpu
from jax.experimental.pallas import tpu_sc as plsc

_B = 16384
_D = 128
_NC = 2
_NS = 16
_NW = _NC * _NS
_BPW = _B // _NW

_mesh = plsc.VectorSubcoreMesh(core_axis_name="c", subcore_axis_name="s")


_CH = 8
_CR = _BPW // _CH
_V = 100
_SR = 8
_SFULL = _V // _SR


@functools.partial(
    pl.kernel,
    out_type=jax.ShapeDtypeStruct((_B, _D), jnp.float32),
    mesh=_mesh,
    scratch_types=[
        pltpu.VMEM((_BPW,), jnp.int32),
        pltpu.VMEM((_CH, _CR, _D), jnp.float32),
        pltpu.VMEM_SHARED((_V, _D), jnp.float32),
        [pltpu.SemaphoreType.DMA] * _CH,
        pltpu.SemaphoreType.DMA,
        pltpu.SemaphoreType.DMA,
    ],
)
def _emb_lookup(idx_hbm, table_hbm, out_hbm, idx_v, rows_v, table_sh, gsems, ssem, isem):
    sid = lax.axis_index("s")
    wid = sid * _NC + lax.axis_index("c")
    base = wid * _BPW
    idx_cp = pltpu.async_copy(idx_hbm.at[pl.ds(base, _BPW)], idx_v, isem)
    @pl.when(sid < _SFULL)
    def _():
        pltpu.sync_copy(
            table_hbm.at[pl.ds(sid * _SR, _SR)], table_sh.at[pl.ds(sid * _SR, _SR)]
        )

    @pl.when(sid == _SFULL)
    def _():
        pltpu.sync_copy(
            table_hbm.at[pl.ds(_SFULL * _SR, _V - _SFULL * _SR)],
            table_sh.at[pl.ds(_SFULL * _SR, _V - _SFULL * _SR)],
        )

    plsc.subcore_barrier()
    idx_cp.wait()
    gathers = [
        pltpu.async_copy(
            table_sh.at[idx_v.at[pl.ds(c * _CR, _CR)]], rows_v.at[c], gsems[c]
        )
        for c in range(_CH)
    ]
    stores = []
    for c in range(_CH):
        gathers[c].wait()
        stores.append(
            pltpu.async_copy(
                rows_v.at[c], out_hbm.at[pl.ds(base + c * _CR, _CR)], ssem
            )
        )
    for cp in stores:
        cp.wait()


def kernel(x, table):
    idx = x.reshape(-1).astype(jnp.int32)
    return _emb_lookup(idx, table)


# speedup vs baseline: 1.6084x; 1.0043x over previous
import functools

import jax
import jax.numpy as jnp
from jax import lax
from jax.experimental import pallas as pl
from jax.experimental.pallas import tpu as pltpu
from jax.experimental.pallas import tpu_sc as plsc

_B = 16384
_D = 128
_NC = 2
_NS = 16
_NW = _NC * _NS
_BPW = _B // _NW

_mesh = plsc.VectorSubcoreMesh(core_axis_name="c", subcore_axis_name="s")


_CH = 8
_CR = _BPW // _CH
_V = 100
_SR = 8
_SFULL = _V // _SR


@functools.partial(
    pl.kernel,
    out_type=jax.ShapeDtypeStruct((_B, _D), jnp.float32),
    mesh=_mesh,
    scratch_types=[
        pltpu.VMEM((_BPW,), jnp.int32),
        pltpu.VMEM((_CH, _CR, _D), jnp.float32),
        pltpu.VMEM_SHARED((_V, _D), jnp.float32),
        [pltpu.SemaphoreType.DMA] * _CH,
        pltpu.SemaphoreType.DMA,
        pltpu.SemaphoreType.DMA,
    ],
    compiler_params=pltpu.CompilerParams(skip_device_barrier=True),
)
def _emb_lookup(idx_hbm, table_hbm, out_hbm, idx_v, rows_v, table_sh, gsems, ssem, isem):
    sid = lax.axis_index("s")
    wid = sid * _NC + lax.axis_index("c")
    base = wid * _BPW
    idx_cp = pltpu.async_copy(idx_hbm.at[pl.ds(base, _BPW)], idx_v, isem)
    @pl.when(sid < _SFULL)
    def _():
        pltpu.sync_copy(
            table_hbm.at[pl.ds(sid * _SR, _SR)], table_sh.at[pl.ds(sid * _SR, _SR)]
        )

    @pl.when(sid == _SFULL)
    def _():
        pltpu.sync_copy(
            table_hbm.at[pl.ds(_SFULL * _SR, _V - _SFULL * _SR)],
            table_sh.at[pl.ds(_SFULL * _SR, _V - _SFULL * _SR)],
        )

    plsc.subcore_barrier()
    idx_cp.wait()
    gathers = [
        pltpu.async_copy(
            table_sh.at[idx_v.at[pl.ds(c * _CR, _CR)]], rows_v.at[c], gsems[c]
        )
        for c in range(_CH)
    ]
    stores = []
    for c in range(_CH):
        gathers[c].wait()
        stores.append(
            pltpu.async_copy(
                rows_v.at[c], out_hbm.at[pl.ds(base + c * _CR, _CR)], ssem
            )
        )
    for cp in stores:
        cp.wait()


def kernel(x, table):
    idx = x.reshape(-1).astype(jnp.int32)
    return _emb_lookup(idx, table)
